# baseline (speedup 1.0000x reference)
import functools

import jax
import jax.numpy as jnp
from jax import lax
from jax.experimental import pallas as pl
from jax.experimental.pallas import tpu as pltpu
from jax.experimental.pallas import tpu_sc as plsc

N = 10000
C = 20000
E = 320000
D = 128
DE = 16
NC_LEN = 60000

NUM_SC = 2
NUM_SUB = 16
NUM_W = NUM_SC * NUM_SUB

C_PAD = 20480
HALF = C_PAD // NUM_SC
ACC_ROWS = HALF + 128
ZROWS_PER_SUB = ACC_ROWS // NUM_SUB
ZBUF_ROWS = 72
DUMMY_ROW = HALF
OUT_ROWS_PER_SUB = HALF // NUM_SUB

A_CHUNK = 96
A_NCHUNKS = NC_LEN // A_CHUNK

E_CHUNK = 128
E_NCHUNKS = E // E_CHUNK
ZERO_ROW2 = 2 * C

_mesh = plsc.VectorSubcoreMesh(core_axis_name="c", subcore_axis_name="s")


@functools.partial(
    pl.kernel,
    mesh=_mesh,
    out_type=jax.ShapeDtypeStruct((C_PAD, D), jnp.float32),
    scratch_types=[
        pltpu.VMEM_SHARED((ACC_ROWS, D), jnp.float32),
        pltpu.VMEM((ZBUF_ROWS, D), jnp.float32),
        pltpu.VMEM((A_CHUNK, D), jnp.float32),
        pltpu.VMEM((A_CHUNK, D), jnp.float32),
        pltpu.VMEM((A_CHUNK,), jnp.int32),
        pltpu.VMEM((A_CHUNK,), jnp.int32),
        pltpu.VMEM((A_CHUNK,), jnp.int32),
        pltpu.VMEM((A_CHUNK,), jnp.int32),
        pltpu.VMEM((A_CHUNK,), jnp.int32),
        pltpu.VMEM((A_CHUNK,), jnp.int32),
        pltpu.SemaphoreType.DMA,
        pltpu.SemaphoreType.DMA,
        pltpu.SemaphoreType.DMA,
        pltpu.SemaphoreType.DMA,
        pltpu.SemaphoreType.DMA,
        pltpu.SemaphoreType.DMA,
        pltpu.SemaphoreType.DMA,
        pltpu.SemaphoreType.DMA,
    ],
)
def _seg_sum(node_hbm, cn_hbm, ci_hbm, out_hbm,
             acc, zbuf, rows0, rows1, nidx0, cidx0, lidx0,
             nidx1, cidx1, lidx1,
             semG0, semG1, semA0, semA1, semN0, semC0, semN1, semC1):
    cid = lax.axis_index("c")
    sid = lax.axis_index("s")

    zv = jnp.zeros((16,), jnp.float32)

    def zrow(i, carry):
        for j in range(D // 16):
            zbuf[i, pl.ds(j * 16, 16)] = zv
        return carry

    lax.fori_loop(0, ZBUF_ROWS, zrow, 0)
    for t in range(ZROWS_PER_SUB // ZBUF_ROWS):
        pltpu.sync_copy(
            zbuf, acc.at[pl.ds(sid * ZROWS_PER_SUB + t * ZBUF_ROWS, ZBUF_ROWS)])
    plsc.subcore_barrier()

    lo = cid * HALF
    nk = (A_NCHUNKS - sid + NUM_SUB - 1) // NUM_SUB
    npairs = nk // 2
    dummy = jnp.full((16,), DUMMY_ROW, jnp.int32)

    def make_lidx(cidx, lidx):
        for j in range(A_CHUNK // 16):
            sl = pl.ds(j * 16, 16)
            loc = cidx[sl] - lo
            valid = (loc >= 0) & (loc < HALF)
            lidx[sl] = jnp.where(valid, loc, dummy)

    def body(k, carry):
        base0 = (sid + NUM_SUB * (2 * k)) * A_CHUNK
        base1 = (sid + NUM_SUB * (2 * k + 1)) * A_CHUNK
        cn0 = pltpu.async_copy(cn_hbm.at[pl.ds(base0, A_CHUNK)], nidx0, semN0)
        cc0 = pltpu.async_copy(ci_hbm.at[pl.ds(base0, A_CHUNK)], cidx0, semC0)
        cn1 = pltpu.async_copy(cn_hbm.at[pl.ds(base1, A_CHUNK)], nidx1, semN1)
        cc1 = pltpu.async_copy(ci_hbm.at[pl.ds(base1, A_CHUNK)], cidx1, semC1)
        cn0.wait()
        g0 = pltpu.async_copy(node_hbm.at[nidx0], rows0, semG0)
        cc0.wait()
        make_lidx(cidx0, lidx0)
        cn1.wait()
        g1 = pltpu.async_copy(node_hbm.at[nidx1], rows1, semG1)
        cc1.wait()
        make_lidx(cidx1, lidx1)
        g0.wait()
        a0 = pltpu.async_copy(rows0, acc.at[lidx0], semA0, add=True)
        g1.wait()
        a1 = pltpu.async_copy(rows1, acc.at[lidx1], semA1, add=True)
        a0.wait()
        a1.wait()
        return carry

    lax.fori_loop(0, npairs, body, 0)

    @pl.when(nk % 2 == 1)
    def _taila():
        base = (sid + NUM_SUB * (nk - 1)) * A_CHUNK
        pltpu.sync_copy(cn_hbm.at[pl.ds(base, A_CHUNK)], nidx0)
        pltpu.sync_copy(ci_hbm.at[pl.ds(base, A_CHUNK)], cidx0)
        pltpu.async_copy(node_hbm.at[nidx0], rows0, semG0).wait()
        make_lidx(cidx0, lidx0)
        pltpu.sync_copy(rows0, acc.at[lidx0], add=True)

    plsc.subcore_barrier()

    pltpu.sync_copy(
        acc.at[pl.ds(sid * OUT_ROWS_PER_SUB, OUT_ROWS_PER_SUB)],
        out_hbm.at[pl.ds(cid * HALF + sid * OUT_ROWS_PER_SUB,
                         OUT_ROWS_PER_SUB)])


@functools.partial(
    pl.kernel,
    mesh=_mesh,
    out_type=jax.ShapeDtypeStruct((E, D), jnp.float32),
    scratch_types=[
        pltpu.VMEM((E_CHUNK, D), jnp.float32),
        pltpu.VMEM((E_CHUNK, D), jnp.float32),
        pltpu.VMEM((E_CHUNK, D), jnp.float32),
        pltpu.VMEM((E_CHUNK, D), jnp.float32),
        pltpu.VMEM((E_CHUNK, D), jnp.float32),
        pltpu.VMEM((E_CHUNK, D), jnp.float32),
        pltpu.VMEM((E_CHUNK,), jnp.int32),
        pltpu.VMEM((E_CHUNK,), jnp.int32),
        pltpu.VMEM((E_CHUNK,), jnp.int32),
        pltpu.VMEM((E_CHUNK,), jnp.int32),
        pltpu.VMEM((E_CHUNK,), jnp.int32),
        pltpu.VMEM((E_CHUNK,), jnp.int32),
        pltpu.VMEM((E_CHUNK,), jnp.int32),
        pltpu.VMEM((E_CHUNK,), jnp.int32),
        pltpu.SemaphoreType.DMA,
        pltpu.SemaphoreType.DMA,
        pltpu.SemaphoreType.DMA,
        pltpu.SemaphoreType.DMA,
        pltpu.SemaphoreType.DMA,
        pltpu.SemaphoreType.DMA,
        pltpu.SemaphoreType.DMA,
        pltpu.SemaphoreType.DMA,
        pltpu.SemaphoreType.DMA,
        pltpu.SemaphoreType.DMA,
    ],
)
def _edge_gather(p2_hbm, s_hbm, r_hbm, contrib_hbm, out_hbm,
                 bufS0, bufR0, bufO0, bufS1, bufR1, bufO1,
                 sraw0, rraw0, srow0, rrow0, sraw1, rraw1, srow1, rrow1,
                 semS0, semR0, semO0, semS1, semR1, semO1,
                 semI0, semI1, semI2, semI3):
    cid = lax.axis_index("c")
    sid = lax.axis_index("s")
    wid = sid * NUM_SC + cid
    nk = (E_NCHUNKS - wid + NUM_W - 1) // NUM_W
    npairs = nk // 2
    zrow = jnp.full((16,), ZERO_ROW2, jnp.int32)

    def rows_from_raw(sraw, rraw, srow, rrow):
        for j in range(E_CHUNK // 16):
            sl = pl.ds(j * 16, 16)
            sv = sraw[sl]
            rv = rraw[sl]
            srow[sl] = sv * 2
            rrow[sl] = jnp.where(sv == rv, zrow, rv * 2 + 1)

    def add_out(bufS, bufR, bufO, base):
        def addrow(i, c2):
            for j in range(D // 16):
                sl = pl.ds(j * 16, 16)
                plsc.addupdate(bufO.at[i, sl], bufS[i, sl] + bufR[i, sl])
            return c2
        lax.fori_loop(0, E_CHUNK, addrow, 0)
        pltpu.sync_copy(bufO, out_hbm.at[pl.ds(base, E_CHUNK)])

    def body(k, carry):
        base0 = (wid + NUM_W * (2 * k)) * E_CHUNK
        base1 = (wid + NUM_W * (2 * k + 1)) * E_CHUNK
        ci0s = pltpu.async_copy(s_hbm.at[pl.ds(base0, E_CHUNK)], sraw0, semI0)
        ci0r = pltpu.async_copy(r_hbm.at[pl.ds(base0, E_CHUNK)], rraw0, semI1)
        ci1s = pltpu.async_copy(s_hbm.at[pl.ds(base1, E_CHUNK)], sraw1, semI2)
        ci1r = pltpu.async_copy(r_hbm.at[pl.ds(base1, E_CHUNK)], rraw1, semI3)
        co0 = pltpu.async_copy(contrib_hbm.at[pl.ds(base0, E_CHUNK)], bufO0,
                               semO0)
        co1 = pltpu.async_copy(contrib_hbm.at[pl.ds(base1, E_CHUNK)], bufO1,
                               semO1)
        ci0s.wait()
        ci0r.wait()
        rows_from_raw(sraw0, rraw0, srow0, rrow0)
        cs0 = pltpu.async_copy(p2_hbm.at[srow0], bufS0, semS0)
        cr0 = pltpu.async_copy(p2_hbm.at[rrow0], bufR0, semR0)
        ci1s.wait()
        ci1r.wait()
        rows_from_raw(sraw1, rraw1, srow1, rrow1)
        cs1 = pltpu.async_copy(p2_hbm.at[srow1], bufS1, semS1)
        cr1 = pltpu.async_copy(p2_hbm.at[rrow1], bufR1, semR1)
        cs0.wait()
        cr0.wait()
        co0.wait()
        add_out(bufS0, bufR0, bufO0, base0)
        cs1.wait()
        cr1.wait()
        co1.wait()
        add_out(bufS1, bufR1, bufO1, base1)
        return carry

    lax.fori_loop(0, npairs, body, 0)

    @pl.when(nk % 2 == 1)
    def _tail():
        base = (wid + NUM_W * (nk - 1)) * E_CHUNK
        pltpu.sync_copy(s_hbm.at[pl.ds(base, E_CHUNK)], sraw0)
        pltpu.sync_copy(r_hbm.at[pl.ds(base, E_CHUNK)], rraw0)
        co = pltpu.async_copy(contrib_hbm.at[pl.ds(base, E_CHUNK)], bufO0,
                              semO0)
        rows_from_raw(sraw0, rraw0, srow0, rrow0)
        cs = pltpu.async_copy(p2_hbm.at[srow0], bufS0, semS0)
        cr = pltpu.async_copy(p2_hbm.at[rrow0], bufR0, semR0)
        cs.wait()
        cr.wait()
        co.wait()
        add_out(bufS0, bufR0, bufO0, base)


def _proj_body(x_ref, w_ref, o_ref):
    o_ref[...] = jnp.dot(x_ref[...], w_ref[...],
                         preferred_element_type=jnp.float32)


_B_BLK = 512

_proj = pl.pallas_call(
    _proj_body,
    grid=(C_PAD // _B_BLK,),
    in_specs=[pl.BlockSpec((_B_BLK, D), lambda i: (i, 0)),
              pl.BlockSpec((D, 2 * D), lambda i: (0, 0))],
    out_specs=pl.BlockSpec((_B_BLK, 2 * D), lambda i: (i, 0)),
    out_shape=jax.ShapeDtypeStruct((C_PAD, 2 * D), jnp.float32),
)


def _contrib_body(ea_ref, we_ref, b_ref, o_ref):
    acc = jnp.dot(ea_ref[...], we_ref[...],
                  preferred_element_type=jnp.float32)
    o_ref[...] = acc + b_ref[...]


_D_BLK = 1280

_contrib = pl.pallas_call(
    _contrib_body,
    grid=(E // _D_BLK,),
    in_specs=[pl.BlockSpec((_D_BLK, DE), lambda i: (i, 0)),
              pl.BlockSpec((DE, D), lambda i: (0, 0)),
              pl.BlockSpec((1, D), lambda i: (0, 0))],
    out_specs=pl.BlockSpec((_D_BLK, D), lambda i: (i, 0)),
    out_shape=jax.ShapeDtypeStruct((E, D), jnp.float32),
)


def kernel(node_attr, edge_attr, cells_node, cells_index,
           cell_edge_index, node_edge_index, W, b):
    Wsr = jnp.concatenate([W[:D], W[D:2 * D]], axis=1)
    We = W[2 * D:]

    contrib = _contrib(edge_attr, We, b.reshape(1, D))
    cell_attr_pad = _seg_sum(node_attr, cells_node, cells_index)
    P = _proj(cell_attr_pad, Wsr)
    P2 = P.reshape(2 * C_PAD, D)

    edge_attr_ = _edge_gather(P2, cell_edge_index[0], cell_edge_index[1],
                              contrib)
    return (node_attr, edge_attr_, node_edge_index, cells_node)


# speedup vs baseline: 1.2023x; 1.2023x over previous
import functools

import jax
import jax.numpy as jnp
from jax import lax
from jax.experimental import pallas as pl
from jax.experimental.pallas import tpu as pltpu
from jax.experimental.pallas import tpu_sc as plsc

N = 10000
C = 20000
E = 320000
D = 128
DE = 16
NC_LEN = 60000

NUM_SC = 2
NUM_SUB = 16
NUM_W = NUM_SC * NUM_SUB

C_PAD = 20480
HALF = C_PAD // NUM_SC
ACC_ROWS = HALF + 128
ZROWS_PER_SUB = ACC_ROWS // NUM_SUB
ZBUF_ROWS = 72
DUMMY_ROW = HALF
OUT_ROWS_PER_SUB = HALF // NUM_SUB

A_CHUNK = 96
A_NCHUNKS = NC_LEN // A_CHUNK

E_CHUNK = 128
E_NCHUNKS = E // E_CHUNK
ZERO_ROW2 = 2 * C

_mesh = plsc.VectorSubcoreMesh(core_axis_name="c", subcore_axis_name="s")


@functools.partial(
    pl.kernel,
    mesh=_mesh,
    out_type=jax.ShapeDtypeStruct((C_PAD, D), jnp.float32),
    scratch_types=[
        pltpu.VMEM_SHARED((ACC_ROWS, D), jnp.float32),
        pltpu.VMEM((ZBUF_ROWS, D), jnp.float32),
        pltpu.VMEM((A_CHUNK, D), jnp.float32),
        pltpu.VMEM((A_CHUNK, D), jnp.float32),
        pltpu.VMEM((A_CHUNK,), jnp.int32),
        pltpu.VMEM((A_CHUNK,), jnp.int32),
        pltpu.VMEM((A_CHUNK,), jnp.int32),
        pltpu.VMEM((A_CHUNK,), jnp.int32),
        pltpu.VMEM((A_CHUNK,), jnp.int32),
        pltpu.VMEM((A_CHUNK,), jnp.int32),
        pltpu.SemaphoreType.DMA,
        pltpu.SemaphoreType.DMA,
        pltpu.SemaphoreType.DMA,
        pltpu.SemaphoreType.DMA,
        pltpu.SemaphoreType.DMA,
        pltpu.SemaphoreType.DMA,
        pltpu.SemaphoreType.DMA,
        pltpu.SemaphoreType.DMA,
    ],
)
def _seg_sum(node_hbm, cn_hbm, ci_hbm, out_hbm,
             acc, zbuf, rows0, rows1, nidx0, cidx0, lidx0,
             nidx1, cidx1, lidx1,
             semG0, semG1, semA0, semA1, semN0, semC0, semN1, semC1):
    cid = lax.axis_index("c")
    sid = lax.axis_index("s")

    zv = jnp.zeros((16,), jnp.float32)

    def zrow(i, carry):
        for j in range(D // 16):
            zbuf[i, pl.ds(j * 16, 16)] = zv
        return carry

    lax.fori_loop(0, ZBUF_ROWS, zrow, 0)
    for t in range(ZROWS_PER_SUB // ZBUF_ROWS):
        pltpu.sync_copy(
            zbuf, acc.at[pl.ds(sid * ZROWS_PER_SUB + t * ZBUF_ROWS, ZBUF_ROWS)])
    plsc.subcore_barrier()

    lo = cid * HALF
    nk = (A_NCHUNKS - sid + NUM_SUB - 1) // NUM_SUB
    npairs = nk // 2
    dummy = jnp.full((16,), DUMMY_ROW, jnp.int32)

    def make_lidx(cidx, lidx):
        for j in range(A_CHUNK // 16):
            sl = pl.ds(j * 16, 16)
            loc = cidx[sl] - lo
            valid = (loc >= 0) & (loc < HALF)
            lidx[sl] = jnp.where(valid, loc, dummy)

    def body(k, carry):
        base0 = (sid + NUM_SUB * (2 * k)) * A_CHUNK
        base1 = (sid + NUM_SUB * (2 * k + 1)) * A_CHUNK
        cn0 = pltpu.async_copy(cn_hbm.at[pl.ds(base0, A_CHUNK)], nidx0, semN0)
        cc0 = pltpu.async_copy(ci_hbm.at[pl.ds(base0, A_CHUNK)], cidx0, semC0)
        cn1 = pltpu.async_copy(cn_hbm.at[pl.ds(base1, A_CHUNK)], nidx1, semN1)
        cc1 = pltpu.async_copy(ci_hbm.at[pl.ds(base1, A_CHUNK)], cidx1, semC1)
        cn0.wait()
        g0 = pltpu.async_copy(node_hbm.at[nidx0], rows0, semG0)
        cc0.wait()
        make_lidx(cidx0, lidx0)
        cn1.wait()
        g1 = pltpu.async_copy(node_hbm.at[nidx1], rows1, semG1)
        cc1.wait()
        make_lidx(cidx1, lidx1)
        g0.wait()
        a0 = pltpu.async_copy(rows0, acc.at[lidx0], semA0, add=True)
        g1.wait()
        a1 = pltpu.async_copy(rows1, acc.at[lidx1], semA1, add=True)
        a0.wait()
        a1.wait()
        return carry

    lax.fori_loop(0, npairs, body, 0)

    @pl.when(nk % 2 == 1)
    def _taila():
        base = (sid + NUM_SUB * (nk - 1)) * A_CHUNK
        pltpu.sync_copy(cn_hbm.at[pl.ds(base, A_CHUNK)], nidx0)
        pltpu.sync_copy(ci_hbm.at[pl.ds(base, A_CHUNK)], cidx0)
        pltpu.async_copy(node_hbm.at[nidx0], rows0, semG0).wait()
        make_lidx(cidx0, lidx0)
        pltpu.sync_copy(rows0, acc.at[lidx0], add=True)

    plsc.subcore_barrier()

    pltpu.sync_copy(
        acc.at[pl.ds(sid * OUT_ROWS_PER_SUB, OUT_ROWS_PER_SUB)],
        out_hbm.at[pl.ds(cid * HALF + sid * OUT_ROWS_PER_SUB,
                         OUT_ROWS_PER_SUB)])


@functools.partial(
    pl.kernel,
    mesh=_mesh,
    out_type=jax.ShapeDtypeStruct((E, D), jnp.float32),
    scratch_types=[
        pltpu.VMEM((E_CHUNK, D), jnp.float32),
        pltpu.VMEM((E_CHUNK, D), jnp.float32),
        pltpu.VMEM((E_CHUNK, D), jnp.float32),
        pltpu.VMEM((E_CHUNK, D), jnp.float32),
        pltpu.VMEM((E_CHUNK, D), jnp.float32),
        pltpu.VMEM((E_CHUNK, D), jnp.float32),
        pltpu.VMEM((E_CHUNK,), jnp.int32),
        pltpu.VMEM((E_CHUNK,), jnp.int32),
        pltpu.VMEM((E_CHUNK,), jnp.int32),
        pltpu.VMEM((E_CHUNK,), jnp.int32),
        pltpu.VMEM((E_CHUNK,), jnp.int32),
        pltpu.VMEM((E_CHUNK,), jnp.int32),
        pltpu.VMEM((E_CHUNK,), jnp.int32),
        pltpu.VMEM((E_CHUNK,), jnp.int32),
        pltpu.SemaphoreType.DMA,
        pltpu.SemaphoreType.DMA,
        pltpu.SemaphoreType.DMA,
        pltpu.SemaphoreType.DMA,
        pltpu.SemaphoreType.DMA,
        pltpu.SemaphoreType.DMA,
        pltpu.SemaphoreType.DMA,
        pltpu.SemaphoreType.DMA,
        pltpu.SemaphoreType.DMA,
        pltpu.SemaphoreType.DMA,
    ],
)
def _edge_gather(p2_hbm, s_hbm, r_hbm, contrib_hbm, out_hbm,
                 bufS0, bufR0, bufO0, bufS1, bufR1, bufO1,
                 sraw0, rraw0, srow0, rrow0, sraw1, rraw1, srow1, rrow1,
                 semS0, semR0, semO0, semS1, semR1, semO1,
                 semI0, semI1, semI2, semI3):
    cid = lax.axis_index("c")
    sid = lax.axis_index("s")
    wid = sid * NUM_SC + cid
    nk = (E_NCHUNKS - wid + NUM_W - 1) // NUM_W
    npairs = nk // 2
    zrow = jnp.full((16,), ZERO_ROW2, jnp.int32)

    def rows_from_raw(sraw, rraw, srow, rrow):
        for j in range(E_CHUNK // 16):
            sl = pl.ds(j * 16, 16)
            sv = sraw[sl]
            rv = rraw[sl]
            srow[sl] = sv * 2
            rrow[sl] = jnp.where(sv == rv, zrow, rv * 2 + 1)

    def add_out(bufS, bufR, bufO, base):
        def addrow(i, c2):
            for j in range(D // 16):
                sl = pl.ds(j * 16, 16)
                plsc.addupdate(bufO.at[i, sl], bufS[i, sl] + bufR[i, sl])
            return c2
        lax.fori_loop(0, E_CHUNK, addrow, 0)
        pltpu.sync_copy(bufO, out_hbm.at[pl.ds(base, E_CHUNK)])

    def body(k, carry):
        base0 = (wid + NUM_W * (2 * k)) * E_CHUNK
        base1 = (wid + NUM_W * (2 * k + 1)) * E_CHUNK
        ci0s = pltpu.async_copy(s_hbm.at[pl.ds(base0, E_CHUNK)], sraw0, semI0)
        ci0r = pltpu.async_copy(r_hbm.at[pl.ds(base0, E_CHUNK)], rraw0, semI1)
        ci1s = pltpu.async_copy(s_hbm.at[pl.ds(base1, E_CHUNK)], sraw1, semI2)
        ci1r = pltpu.async_copy(r_hbm.at[pl.ds(base1, E_CHUNK)], rraw1, semI3)
        co0 = pltpu.async_copy(contrib_hbm.at[pl.ds(base0, E_CHUNK)], bufO0,
                               semO0)
        co1 = pltpu.async_copy(contrib_hbm.at[pl.ds(base1, E_CHUNK)], bufO1,
                               semO1)
        ci0s.wait()
        ci0r.wait()
        rows_from_raw(sraw0, rraw0, srow0, rrow0)
        cs0 = pltpu.async_copy(p2_hbm.at[srow0], bufS0, semS0)
        cr0 = pltpu.async_copy(p2_hbm.at[rrow0], bufR0, semR0)
        ci1s.wait()
        ci1r.wait()
        rows_from_raw(sraw1, rraw1, srow1, rrow1)
        cs1 = pltpu.async_copy(p2_hbm.at[srow1], bufS1, semS1)
        cr1 = pltpu.async_copy(p2_hbm.at[rrow1], bufR1, semR1)
        cs0.wait()
        cr0.wait()
        co0.wait()
        add_out(bufS0, bufR0, bufO0, base0)
        cs1.wait()
        cr1.wait()
        co1.wait()
        add_out(bufS1, bufR1, bufO1, base1)
        return carry

    lax.fori_loop(0, npairs, body, 0)

    @pl.when(nk % 2 == 1)
    def _tail():
        base = (wid + NUM_W * (nk - 1)) * E_CHUNK
        pltpu.sync_copy(s_hbm.at[pl.ds(base, E_CHUNK)], sraw0)
        pltpu.sync_copy(r_hbm.at[pl.ds(base, E_CHUNK)], rraw0)
        co = pltpu.async_copy(contrib_hbm.at[pl.ds(base, E_CHUNK)], bufO0,
                              semO0)
        rows_from_raw(sraw0, rraw0, srow0, rrow0)
        cs = pltpu.async_copy(p2_hbm.at[srow0], bufS0, semS0)
        cr = pltpu.async_copy(p2_hbm.at[rrow0], bufR0, semR0)
        cs.wait()
        cr.wait()
        co.wait()
        add_out(bufS0, bufR0, bufO0, base)


def _tc_body(x_ref, wsr_ref, ea_ref, we_ref, b_ref, p_ref, c_ref):
    p_ref[...] = jnp.dot(x_ref[...], wsr_ref[...],
                         preferred_element_type=jnp.float32)
    acc = jnp.dot(ea_ref[...], we_ref[...],
                  preferred_element_type=jnp.float32)
    c_ref[...] = acc + b_ref[...]


_B_GRID = 40
_B_BLK = C_PAD // _B_GRID
_D_BLK = E // _B_GRID

_tc_stage = pl.pallas_call(
    _tc_body,
    grid=(_B_GRID,),
    in_specs=[pl.BlockSpec((_B_BLK, D), lambda i: (i, 0)),
              pl.BlockSpec((D, 2 * D), lambda i: (0, 0)),
              pl.BlockSpec((_D_BLK, DE), lambda i: (i, 0)),
              pl.BlockSpec((DE, D), lambda i: (0, 0)),
              pl.BlockSpec((1, D), lambda i: (0, 0))],
    out_specs=[pl.BlockSpec((_B_BLK, 2 * D), lambda i: (i, 0)),
               pl.BlockSpec((_D_BLK, D), lambda i: (i, 0))],
    out_shape=(jax.ShapeDtypeStruct((C_PAD, 2 * D), jnp.float32),
               jax.ShapeDtypeStruct((E, D), jnp.float32)),
)


def kernel(node_attr, edge_attr, cells_node, cells_index,
           cell_edge_index, node_edge_index, W, b):
    Wsr = jnp.concatenate([W[:D], W[D:2 * D]], axis=1)
    We = W[2 * D:]

    cell_attr_pad = _seg_sum(node_attr, cells_node, cells_index)
    P, contrib = _tc_stage(cell_attr_pad, Wsr, edge_attr, We,
                           b.reshape(1, D))
    P2 = P.reshape(2 * C_PAD, D)

    edge_attr_ = _edge_gather(P2, cell_edge_index[0], cell_edge_index[1],
                              contrib)
    return (node_attr, edge_attr_, node_edge_index, cells_node)
